# baseline (speedup 1.0000x reference)
import dataclasses
import functools

import jax
import jax.numpy as jnp
from jax import lax
from jax.experimental import pallas as pl
from jax.experimental.pallas import tpu as pltpu
from jax.experimental.pallas import tpu_sc as plsc

N = 10000
F = 128
C = 16
E = 320000
NC = 2
NS = 16
NW = NC * NS
W = 128
GCH = E // W
TSTEPS = (GCH + NW - 1) // NW
NSLOT = 4
TLOOP = ((TSTEPS + NSLOT - 1) // NSLOT) * NSLOT
FAKE0 = NW * TLOOP * W - E

B0 = 632
BL = N - (NS - 1) * B0

_mesh = plsc.VectorSubcoreMesh(core_axis_name="c", subcore_axis_name="s")

_cp = pltpu.CompilerParams()
if "needs_layout_passes" in pltpu.CompilerParams.__dataclass_fields__:
    _cp = dataclasses.replace(_cp, needs_layout_passes=False)


def _part_init(zeros_hbm, acc_sh, s):
    base = pl.multiple_of(s * B0, 8)

    @pl.when(s < NS - 1)
    def _():
        pltpu.sync_copy(zeros_hbm, acc_sh.at[pl.ds(base, B0)])

    @pl.when(s == NS - 1)
    def _():
        pltpu.sync_copy(zeros_hbm.at[pl.ds(0, BL)], acc_sh.at[pl.ds(base, BL)])


def _part_copyout(acc_sh, out_hbm, c, s):
    base = pl.multiple_of(s * B0, 8)

    @pl.when(s < NS - 1)
    def _():
        pltpu.sync_copy(acc_sh.at[pl.ds(base, B0)],
                        out_hbm.at[c, pl.ds(base, B0)])

    @pl.when(s == NS - 1)
    def _():
        pltpu.sync_copy(acc_sh.at[pl.ds(base, BL)],
                        out_hbm.at[c, pl.ds(base, BL)])


@functools.partial(
    pl.kernel, mesh=_mesh,
    out_type=jax.ShapeDtypeStruct((NW, N), jnp.float32),
    compiler_params=_cp,
    scratch_types=[
        pltpu.VMEM((TLOOP, W), jnp.int32),
        pltpu.VMEM((N,), jnp.float32),
        pltpu.SemaphoreType.DMA,
    ],
)
def _sc_degree(ei_hbm, out_hbm, stage_v, hist_v, dsem):
    c = lax.axis_index("c")
    s = lax.axis_index("s")
    wid = s * NC + c

    zeros16 = jnp.zeros((16,), jnp.int32)
    for r in (TLOOP - 2, TLOOP - 1):
        for j in range(W // 16):
            stage_v[r, pl.ds(j * 16, 16)] = zeros16

    @pl.loop(0, TLOOP)
    def _(t):
        g = t * NW + wid

        @pl.when(g < GCH)
        def _():
            off = pl.multiple_of(g * W, 128)
            pltpu.async_copy(ei_hbm.at[1, pl.ds(off, W)], stage_v.at[t], dsem)

    @pl.loop(0, N // 16)
    def _(i):
        hist_v[pl.ds(i * 16, 16)] = jnp.zeros((16,), jnp.float32)

    @pl.loop(0, TLOOP)
    def _(t):
        @pl.when(t * NW + wid < GCH)
        def _():
            pltpu.make_async_copy(ei_hbm.at[1, pl.ds(0, W)],
                                  stage_v.at[t], dsem).wait()

    ones16 = jnp.ones((16,), jnp.float32)

    @pl.loop(0, TLOOP)
    def _(t):
        for j in range(W // 16):
            idx16 = stage_v[t, pl.ds(j * 16, 16)]
            plsc.addupdate_scatter(hist_v, [idx16], ones16)

    pltpu.sync_copy(hist_v, out_hbm.at[wid])


@functools.partial(
    pl.kernel, mesh=_mesh,
    out_type=jax.ShapeDtypeStruct((NC, N, F), jnp.float32),
    scratch_types=[
        pltpu.VMEM((NSLOT, W), jnp.int32),
        pltpu.VMEM((NSLOT, W), jnp.int32),
        pltpu.VMEM((2 * W, F), jnp.float32),
        pltpu.VMEM_SHARED((N, F), jnp.float32),
        pltpu.SemaphoreType.DMA,
        pltpu.SemaphoreType.DMA,
        pltpu.SemaphoreType.DMA,
        pltpu.SemaphoreType.DMA,
        pltpu.SemaphoreType.DMA,
        pltpu.SemaphoreType.DMA,
        pltpu.SemaphoreType.DMA,
        pltpu.SemaphoreType.DMA,
        pltpu.SemaphoreType.DMA,
        pltpu.SemaphoreType.DMA,
        pltpu.SemaphoreType.DMA,
        pltpu.SemaphoreType.DMA,
    ],
)
def _sc_prop(ei_hbm, vals_hbm, zeros_hbm, out_hbm,
             sidx_v, didx_v, rows_v, acc_sh,
             gs0, gs1, ss0, ss1, si0, si1, si2, si3, di0, di1, di2, di3):
    c = lax.axis_index("c")
    s = lax.axis_index("s")
    wid = s * NC + c
    gsem = (gs0, gs1)
    ssem = (ss0, ss1)
    sisem = (si0, si1, si2, si3)
    disem = (di0, di1, di2, di3)

    def load_idx(row, t, q, slots, sems):
        off = pl.multiple_of((t * NW + wid) * W, 128)
        pltpu.async_copy(ei_hbm.at[row, pl.ds(off, W)], slots.at[q], sems[q])

    def gather(q, half):
        pltpu.async_copy(vals_hbm.at[sidx_v.at[q]],
                         rows_v.at[pl.ds(half * W, W)], gsem[half])

    def scatter(q, half):
        pltpu.async_copy(rows_v.at[pl.ds(half * W, W)],
                         acc_sh.at[didx_v.at[q]], ssem[half], add=True)

    def drain_rows(sems, half):
        pltpu.make_async_copy(vals_hbm.at[pl.ds(0, W)],
                              rows_v.at[pl.ds(half * W, W)],
                              sems[half]).wait()

    def drain_slot(slots, sems, q):
        pltpu.make_async_copy(ei_hbm.at[0, pl.ds(0, W)], slots.at[q],
                              sems[q]).wait()

    for q in range(NSLOT):
        load_idx(0, q, q, sidx_v, sisem)
        load_idx(1, q, q, didx_v, disem)
    drain_slot(sidx_v, sisem, 0)
    gather(0, 0)
    _part_init(zeros_hbm, acc_sh, s)
    plsc.subcore_barrier()

    def step(k, q, guarded):
        half = q % 2
        t = NSLOT * k + q
        g = t * NW + wid

        def when(cond, fn):
            if guarded:
                pl.when(cond)(fn)
            else:
                fn()

        def _next():
            drain_slot(sidx_v, sisem, (q + 1) % NSLOT)
            gather((q + 1) % NSLOT, 1 - half)

        def _cur_g():
            drain_rows(gsem, half)

        def _load_s():
            load_idx(0, t + NSLOT, q, sidx_v, sisem)

        def _cur_s():
            drain_slot(didx_v, disem, q)
            scatter(q, half)
            drain_rows(ssem, half)

        def _load_d():
            load_idx(1, t + NSLOT, q, didx_v, disem)

        when(g + NW < GCH, _next)
        when(g < GCH, _cur_g)
        when(g + NSLOT * NW < GCH, _load_s)
        when(g < GCH, _cur_s)
        when(g + NSLOT * NW < GCH, _load_d)

    KMAIN = (TSTEPS - NSLOT - 1) // NSLOT

    @pl.loop(0, KMAIN)
    def _(k):
        for q in range(NSLOT):
            step(k, q, False)

    @pl.loop(0, TLOOP // NSLOT - KMAIN)
    def _(k2):
        for q in range(NSLOT):
            step(k2 + KMAIN, q, True)

    plsc.subcore_barrier()
    _part_copyout(acc_sh, out_hbm, c, s)


def _tc1_body(degp_ref, x_ref, w1_ref, xw1p_ref, dis_ref):
    dsum = jnp.sum(degp_ref[...], axis=0)[:, None]
    row = lax.broadcasted_iota(jnp.int32, (N, 1), 0)
    deg = dsum - jnp.where(row == 0, jnp.float32(FAKE0), 0.0) + 1.0
    dis = 1.0 / jnp.sqrt(deg)
    xw1 = jnp.dot(x_ref[...], w1_ref[...], preferred_element_type=jnp.float32)
    xw1p_ref[...] = dis * xw1
    dis_ref[...] = dis


def _tc2_body(acc_ref, xw1p_ref, dis_ref, b1_ref, hp_ref):
    t = dis_ref[...] * (acc_ref[0] + acc_ref[1] + xw1p_ref[...]) + b1_ref[...]
    hp_ref[...] = dis_ref[...] * jnp.maximum(t, 0.0)


def _tc3_body(acc_ref, hp_ref, dis_ref, w2_ref, b2_ref, out_ref):
    g = dis_ref[...] * (acc_ref[0] + acc_ref[1] + hp_ref[...])
    out_ref[...] = (jnp.dot(g, w2_ref[...],
                            preferred_element_type=jnp.float32) + b2_ref[...])


def kernel(x, edge_index, W1, b1, W2, b2):
    x = x.astype(jnp.float32)
    ei = edge_index.astype(jnp.int32)
    zeros_f = jnp.zeros((B0, F), jnp.float32)

    degp = _sc_degree(ei)

    xw1p, dis = pl.pallas_call(
        _tc1_body,
        out_shape=[
            jax.ShapeDtypeStruct((N, F), jnp.float32),
            jax.ShapeDtypeStruct((N, 1), jnp.float32),
        ],
    )(degp, x, W1)

    acc1 = _sc_prop(ei, xw1p, zeros_f)

    hp = pl.pallas_call(
        _tc2_body,
        out_shape=jax.ShapeDtypeStruct((N, F), jnp.float32),
    )(acc1, xw1p, dis, b1)

    acc2 = _sc_prop(ei, hp, zeros_f)

    out = pl.pallas_call(
        _tc3_body,
        out_shape=jax.ShapeDtypeStruct((N, C), jnp.float32),
    )(acc2, hp, dis, W2, b2)

    return out


# speedup vs baseline: 1.0014x; 1.0014x over previous
import dataclasses
import functools

import jax
import jax.numpy as jnp
from jax import lax
from jax.experimental import pallas as pl
from jax.experimental.pallas import tpu as pltpu
from jax.experimental.pallas import tpu_sc as plsc

N = 10000
F = 128
C = 16
E = 320000
NC = 2
NS = 16
NW = NC * NS
W = 128
GCH = E // W
TSTEPS = (GCH + NW - 1) // NW
NSLOT = 4
TLOOP = ((TSTEPS + NSLOT - 1) // NSLOT) * NSLOT
FAKE0 = NW * TLOOP * W - E

B0 = 632
BL = N - (NS - 1) * B0

_mesh = plsc.VectorSubcoreMesh(core_axis_name="c", subcore_axis_name="s")

_cp = pltpu.CompilerParams()
if "needs_layout_passes" in pltpu.CompilerParams.__dataclass_fields__:
    _cp = dataclasses.replace(_cp, needs_layout_passes=False)


def _part_init(zeros_hbm, acc_sh, s):
    base = pl.multiple_of(s * B0, 8)

    @pl.when(s < NS - 1)
    def _():
        pltpu.sync_copy(zeros_hbm, acc_sh.at[pl.ds(base, B0)])

    @pl.when(s == NS - 1)
    def _():
        pltpu.sync_copy(zeros_hbm.at[pl.ds(0, BL)], acc_sh.at[pl.ds(base, BL)])


def _part_copyout(acc_sh, out_hbm, c, s):
    base = pl.multiple_of(s * B0, 8)

    @pl.when(s < NS - 1)
    def _():
        pltpu.sync_copy(acc_sh.at[pl.ds(base, B0)],
                        out_hbm.at[c, pl.ds(base, B0)])

    @pl.when(s == NS - 1)
    def _():
        pltpu.sync_copy(acc_sh.at[pl.ds(base, BL)],
                        out_hbm.at[c, pl.ds(base, BL)])


@functools.partial(
    pl.kernel, mesh=_mesh,
    out_type=jax.ShapeDtypeStruct((NW, N), jnp.float32),
    compiler_params=_cp,
    scratch_types=[
        pltpu.VMEM((TLOOP, W), jnp.int32),
        pltpu.VMEM((N,), jnp.float32),
        pltpu.SemaphoreType.DMA,
    ],
)
def _sc_degree(ei_hbm, out_hbm, stage_v, hist_v, dsem):
    c = lax.axis_index("c")
    s = lax.axis_index("s")
    wid = s * NC + c

    zeros16 = jnp.zeros((16,), jnp.int32)
    for r in (TLOOP - 2, TLOOP - 1):
        for j in range(W // 16):
            stage_v[r, pl.ds(j * 16, 16)] = zeros16

    @pl.loop(0, TLOOP)
    def _(t):
        g = t * NW + wid

        @pl.when(g < GCH)
        def _():
            off = pl.multiple_of(g * W, 128)
            pltpu.async_copy(ei_hbm.at[1, pl.ds(off, W)], stage_v.at[t], dsem)

    @pl.loop(0, N // 16)
    def _(i):
        hist_v[pl.ds(i * 16, 16)] = jnp.zeros((16,), jnp.float32)

    @pl.loop(0, TLOOP)
    def _(t):
        @pl.when(t * NW + wid < GCH)
        def _():
            pltpu.make_async_copy(ei_hbm.at[1, pl.ds(0, W)],
                                  stage_v.at[t], dsem).wait()

    ones16 = jnp.ones((16,), jnp.float32)

    @pl.loop(0, TLOOP)
    def _(t):
        for j in range(W // 16):
            idx16 = stage_v[t, pl.ds(j * 16, 16)]
            plsc.addupdate_scatter(hist_v, [idx16], ones16)

    pltpu.sync_copy(hist_v, out_hbm.at[wid])


@functools.partial(
    pl.kernel, mesh=_mesh,
    out_type=jax.ShapeDtypeStruct((NC, N, F), jnp.float32),
    scratch_types=[
        pltpu.VMEM((NSLOT, W), jnp.int32),
        pltpu.VMEM((NSLOT, W), jnp.int32),
        pltpu.VMEM((2 * W, F), jnp.float32),
        pltpu.VMEM_SHARED((N, F), jnp.float32),
        pltpu.SemaphoreType.DMA,
        pltpu.SemaphoreType.DMA,
        pltpu.SemaphoreType.DMA,
        pltpu.SemaphoreType.DMA,
        pltpu.SemaphoreType.DMA,
        pltpu.SemaphoreType.DMA,
        pltpu.SemaphoreType.DMA,
        pltpu.SemaphoreType.DMA,
        pltpu.SemaphoreType.DMA,
        pltpu.SemaphoreType.DMA,
        pltpu.SemaphoreType.DMA,
        pltpu.SemaphoreType.DMA,
    ],
)
def _sc_prop(ei_hbm, vals_hbm, zeros_hbm, out_hbm,
             sidx_v, didx_v, rows_v, acc_sh,
             gs0, gs1, ss0, ss1, si0, si1, si2, si3, di0, di1, di2, di3):
    c = lax.axis_index("c")
    s = lax.axis_index("s")
    wid = s * NC + c
    gsem = (gs0, gs1)
    ssem = (ss0, ss1)
    sisem = (si0, si1, si2, si3)
    disem = (di0, di1, di2, di3)

    def load_idx(row, t, q, slots, sems):
        off = pl.multiple_of((t * NW + wid) * W, 128)
        pltpu.async_copy(ei_hbm.at[row, pl.ds(off, W)], slots.at[q], sems[q])

    def gather(q, half):
        pltpu.async_copy(vals_hbm.at[sidx_v.at[q]],
                         rows_v.at[pl.ds(half * W, W)], gsem[half])

    def scatter(q, half):
        pltpu.async_copy(rows_v.at[pl.ds(half * W, W)],
                         acc_sh.at[didx_v.at[q]], ssem[half], add=True)

    def drain_rows(sems, half):
        pltpu.make_async_copy(vals_hbm.at[pl.ds(0, W)],
                              rows_v.at[pl.ds(half * W, W)],
                              sems[half]).wait()

    def drain_slot(slots, sems, q):
        pltpu.make_async_copy(ei_hbm.at[0, pl.ds(0, W)], slots.at[q],
                              sems[q]).wait()

    for q in range(NSLOT):
        load_idx(0, q, q, sidx_v, sisem)
        load_idx(1, q, q, didx_v, disem)
    drain_slot(sidx_v, sisem, 0)
    gather(0, 0)
    _part_init(zeros_hbm, acc_sh, s)
    plsc.subcore_barrier()

    @pl.loop(0, TLOOP // NSLOT)
    def _(k):
        for q in range(NSLOT):
            half = q % 2
            t = NSLOT * k + q
            g = t * NW + wid

            @pl.when(g + NW < GCH)
            def _():
                drain_slot(sidx_v, sisem, (q + 1) % NSLOT)
                gather((q + 1) % NSLOT, 1 - half)

            @pl.when(g < GCH)
            def _():
                drain_rows(gsem, half)

            @pl.when(g + NSLOT * NW < GCH)
            def _():
                load_idx(0, t + NSLOT, q, sidx_v, sisem)

            @pl.when(g < GCH)
            def _():
                drain_slot(didx_v, disem, q)
                scatter(q, half)
                drain_rows(ssem, half)

            @pl.when(g + NSLOT * NW < GCH)
            def _():
                load_idx(1, t + NSLOT, q, didx_v, disem)

    plsc.subcore_barrier()
    _part_copyout(acc_sh, out_hbm, c, s)


def _tc1_body(degp_ref, x_ref, w1_ref, xw1p_ref, dis_ref):
    dsum = jnp.sum(degp_ref[...], axis=0)[:, None]
    row = lax.broadcasted_iota(jnp.int32, (N, 1), 0)
    deg = dsum - jnp.where(row == 0, jnp.float32(FAKE0), 0.0) + 1.0
    dis = 1.0 / jnp.sqrt(deg)
    xw1 = jnp.dot(x_ref[...], w1_ref[...], preferred_element_type=jnp.float32)
    xw1p_ref[...] = dis * xw1
    dis_ref[...] = dis


def _tc2_body(acc_ref, xw1p_ref, dis_ref, b1_ref, hp_ref):
    t = dis_ref[...] * (acc_ref[0] + acc_ref[1] + xw1p_ref[...]) + b1_ref[...]
    hp_ref[...] = dis_ref[...] * jnp.maximum(t, 0.0)


def _tc3_body(acc_ref, hp_ref, dis_ref, w2_ref, b2_ref, out_ref):
    g = dis_ref[...] * (acc_ref[0] + acc_ref[1] + hp_ref[...])
    out_ref[...] = (jnp.dot(g, w2_ref[...],
                            preferred_element_type=jnp.float32) + b2_ref[...])


def kernel(x, edge_index, W1, b1, W2, b2):
    x = x.astype(jnp.float32)
    ei = edge_index.astype(jnp.int32)
    zeros_f = jnp.zeros((B0, F), jnp.float32)

    degp = _sc_degree(ei)

    xw1p, dis = pl.pallas_call(
        _tc1_body,
        out_shape=[
            jax.ShapeDtypeStruct((N, F), jnp.float32),
            jax.ShapeDtypeStruct((N, 1), jnp.float32),
        ],
    )(degp, x, W1)

    acc1 = _sc_prop(ei, xw1p, zeros_f)

    hp = pl.pallas_call(
        _tc2_body,
        out_shape=jax.ShapeDtypeStruct((N, F), jnp.float32),
    )(acc1, xw1p, dis, b1)

    acc2 = _sc_prop(ei, hp, zeros_f)

    out = pl.pallas_call(
        _tc3_body,
        out_shape=jax.ShapeDtypeStruct((N, C), jnp.float32),
    )(acc2, hp, dis, W2, b2)

    return out
